# Initial kernel scaffold; baseline (speedup 1.0000x reference)
import jax
import jax.numpy as jnp
from jax.experimental import pallas as pl


def kernel(raw_inputs, embedding_table):
    raise NotImplementedError("write your pallas kernel here")


# speedup vs baseline: 7.7663x; 7.7663x over previous
import functools

import jax
import jax.numpy as jnp
from jax import lax
from jax.experimental import pallas as pl
from jax.experimental.pallas import tpu as pltpu
from jax.experimental.pallas import tpu_sc as plsc

B_ = 8
N_ = 512
H_ = 16
T_ = B_ * N_ * N_
MAXD = 510
ZROW = MAXD + 2

_INFO = plsc.get_sparse_core_info()
NC = _INFO.num_cores
NS = _INFO.num_subcores
NW = NC * NS
PER_W = T_ // NW

SUB = 128
NSUB = 16
CHUNK = SUB * NSUB
NCHUNK = PER_W // CHUNK
LANES = 16


def _body(idx_hbm, tab_hbm, out_hbm, idx_v, rows_v, sem):
    wid = lax.axis_index("s") * NC + lax.axis_index("c")
    row0 = wid * (PER_W // SUB)

    def chunk(k, carry):
        r0 = row0 + k * NSUB
        pltpu.sync_copy(idx_hbm.at[pl.ds(r0, NSUB), :], idx_v)
        for j in range(NSUB):
            for i in range(SUB // LANES):
                iv = idx_v[j, pl.ds(i * LANES, LANES)]
                cl = jnp.minimum(jnp.maximum(iv, 0), MAXD + 1)
                idx_v[j, pl.ds(i * LANES, LANES)] = jnp.where(iv < 0, ZROW, cl)
        cps = []
        for j in range(NSUB):
            cps.append(pltpu.async_copy(
                tab_hbm.at[idx_v.at[j]],
                rows_v.at[pl.ds(j * SUB, SUB), :],
                sem))
        for cp in cps:
            cp.wait()
        pltpu.sync_copy(rows_v, out_hbm.at[pl.ds(r0 * SUB, CHUNK), :])
        return carry

    lax.fori_loop(0, NCHUNK, chunk, 0)


def kernel(raw_inputs, embedding_table):
    idx2d = raw_inputs.astype(jnp.int32).reshape(T_ // SUB, SUB)
    tab_aug = jnp.concatenate(
        [embedding_table, jnp.zeros((1, H_), jnp.float32)], axis=0)
    mesh = plsc.VectorSubcoreMesh(core_axis_name="c", subcore_axis_name="s")
    run = functools.partial(
        pl.kernel,
        mesh=mesh,
        out_type=jax.ShapeDtypeStruct((T_, H_), jnp.float32),
        scratch_types=[
            pltpu.VMEM((NSUB, SUB), jnp.int32),
            pltpu.VMEM((CHUNK, H_), jnp.float32),
            pltpu.SemaphoreType.DMA,
        ],
        compiler_params=pltpu.CompilerParams(use_tc_tiling_on_sc=False),
    )(_body)
    out = run(idx2d, tab_aug)
    return out.reshape(B_, N_, N_, H_)
